# Initial kernel scaffold; baseline (speedup 1.0000x reference)
import jax
import jax.numpy as jnp
from jax.experimental import pallas as pl


def kernel(features, multi_r_data, batch_nodes, W0, b0, W1, b1):
    raise NotImplementedError("write your pallas kernel here")


# speedup vs baseline: 11.9195x; 11.9195x over previous
import functools

import jax
import jax.numpy as jnp
from jax import lax
from jax.experimental import pallas as pl
from jax.experimental.pallas import tpu as pltpu
from jax.experimental.pallas import tpu_sc as plsc

N = 50000
NP = 50048
E = 800000
NCH = 391
TRASH = 50000
HID = 64
DT = 0.5
C1 = DT * DT / 2.0
C2 = DT * DT
NSUB = 16
BM = 3128
R = 2
B = 4096
GRP = 23
NG = 17
NBUF = 4
LEAD = 2

_mesh = plsc.VectorSubcoreMesh(core_axis_name="c", subcore_axis_name="s")
_sc_params = pltpu.CompilerParams(use_tc_tiling_on_sc=False)


def _prep_body(mr_ref, colr_ref, rows_ref, rowg_ref):
    r = pl.program_id(0)
    row = mr_ref[0, 0]
    col = mr_ref[0, 1]
    self_m = row == col
    colr_ref[0] = jnp.where(self_m, TRASH, col)
    rows_ref[0] = jnp.where(self_m, row, TRASH)
    rowg_ref[0] = row + r * NP


def _prep(mr):
    out = jax.ShapeDtypeStruct((R, 6250, 128), jnp.int32)
    return pl.pallas_call(
        _prep_body,
        grid=(R,),
        in_specs=[pl.BlockSpec((1, 2, 6250, 128), lambda r: (r, 0, 0, 0))],
        out_specs=[pl.BlockSpec((1, 6250, 128), lambda r: (r, 0, 0))] * 3,
        out_shape=[out, out, out],
    )(mr)


def _deg_body(colr, rows, zer, deg_out, self_out, dacc, sacc, ones_v,
              icb, irb, dbuf, dsem, ssem):
    c = lax.axis_index("c")
    s = lax.axis_index("s")
    for q in range(8):
        ones_v[pl.ds(q * 16, 16)] = jnp.full((16,), 1.0, jnp.float32)
    pltpu.sync_copy(zer, dbuf)
    pltpu.sync_copy(dbuf, dacc.at[pl.ds(s * BM, BM)])
    pltpu.sync_copy(dbuf, sacc.at[pl.ds(s * BM, BM)])
    plsc.subcore_barrier()
    wrow = (c * NSUB + s) * NCH

    def group(g, carry):
        pltpu.sync_copy(colr.at[pl.ds(wrow + g * GRP, GRP)], icb)
        pltpu.sync_copy(rows.at[pl.ds(wrow + g * GRP, GRP)], irb)
        for k in range(GRP):
            pltpu.async_copy(ones_v, dacc.at[icb.at[k]], dsem, add=True)
            pltpu.async_copy(ones_v, sacc.at[irb.at[k]], ssem, add=True)
        for k in range(GRP):
            pltpu.make_async_copy(ones_v, dacc.at[icb.at[k]], dsem).wait()
            pltpu.make_async_copy(ones_v, sacc.at[irb.at[k]], ssem).wait()
        return carry

    lax.fori_loop(0, NG, group, 0)
    plsc.subcore_barrier()
    pltpu.sync_copy(dacc.at[pl.ds(s * BM, BM)], dbuf)
    pltpu.sync_copy(dbuf, deg_out.at[pl.ds(c * NP + s * BM, BM)])
    pltpu.sync_copy(sacc.at[pl.ds(s * BM, BM)], dbuf)
    pltpu.sync_copy(dbuf, self_out.at[pl.ds(c * NP + s * BM, BM)])


_deg = functools.partial(
    pl.kernel,
    out_type=[jax.ShapeDtypeStruct((R * NP,), jnp.float32)] * 2,
    mesh=_mesh,
    compiler_params=_sc_params,
    scratch_types=[
        pltpu.VMEM_SHARED((NP,), jnp.float32),
        pltpu.VMEM_SHARED((NP,), jnp.float32),
        pltpu.VMEM((128,), jnp.float32),
        pltpu.VMEM((GRP, 128), jnp.int32),
        pltpu.VMEM((GRP, 128), jnp.int32),
        pltpu.VMEM((BM,), jnp.float32),
        pltpu.SemaphoreType.DMA,
        pltpu.SemaphoreType.DMA,
    ],
)(_deg_body)


def _mm_body(feat, w0, bb0, w1, bb1, deg, selfc,
             phi0_o, phi1_o, yq0_o, yq1_o, yq2_o, yq3_o, dis_o, diag_o):
    f = feat[...]
    p0 = jnp.dot(f, w0[0], preferred_element_type=jnp.float32) + bb0[0]
    p1 = jnp.dot(f, w1[0], preferred_element_type=jnp.float32) + bb1[0]
    dg = deg[0]
    pos = dg > 0.0
    safe = jnp.maximum(dg, 1.0)
    dis = jnp.where(pos, lax.rsqrt(safe), 0.0)
    loop = jnp.where(selfc[0] > 0.0, 1.0, 0.0)
    diag = jnp.where(pos, loop / safe, 0.0)
    phi0_o[0] = p0
    phi1_o[0] = p1
    y0 = p0 * dis
    for q, ref in enumerate((yq0_o, yq1_o, yq2_o, yq3_o)):
        ref[0] = y0[:, q * 16:(q + 1) * 16]
    dis_o[0] = dis
    diag_o[0] = diag


def _mm(featp, W0, b0, W1, b1, deg, selfc):
    f64 = jax.ShapeDtypeStruct((R, NP, HID), jnp.float32)
    f16o = jax.ShapeDtypeStruct((R, NP, 16), jnp.float32)
    f1 = jax.ShapeDtypeStruct((R, NP, 1), jnp.float32)
    bs16 = pl.BlockSpec((1, BM, 16), lambda r, i: (r, i, 0))
    return pl.pallas_call(
        _mm_body,
        grid=(R, NSUB),
        in_specs=[
            pl.BlockSpec((BM, 128), lambda r, i: (i, 0)),
            pl.BlockSpec((1, 128, HID), lambda r, i: (r, 0, 0)),
            pl.BlockSpec((1, 1, HID), lambda r, i: (r, 0, 0)),
            pl.BlockSpec((1, 128, HID), lambda r, i: (r, 0, 0)),
            pl.BlockSpec((1, 1, HID), lambda r, i: (r, 0, 0)),
            pl.BlockSpec((1, BM, 1), lambda r, i: (r, i, 0)),
            pl.BlockSpec((1, BM, 1), lambda r, i: (r, i, 0)),
        ],
        out_specs=[
            pl.BlockSpec((1, BM, HID), lambda r, i: (r, i, 0)),
            pl.BlockSpec((1, BM, HID), lambda r, i: (r, i, 0)),
            bs16, bs16, bs16, bs16,
            pl.BlockSpec((1, BM, 1), lambda r, i: (r, i, 0)),
            pl.BlockSpec((1, BM, 1), lambda r, i: (r, i, 0)),
        ],
        out_shape=[f64, f64, f16o, f16o, f16o, f16o, f1, f1],
    )(featp, W0, b0, W1, b1, deg, selfc)


def _prop_body(yq0, yq1, yq2, yq3, rowg, colr, zrows, z_out,
               acc, igb, iscb, rb0, rb1, rb2, rb3, zbuf,
               gs0, gs1, gs2, gs3, ss0, ss1, ss2, ss3):
    c = lax.axis_index("c")
    s = lax.axis_index("s")
    rbufs = (rb0, rb1, rb2, rb3)
    gsems = (gs0, gs1, gs2, gs3)
    ssems = (ss0, ss1, ss2, ss3)
    for r in range(R):
        for ph in range(2):
            pltpu.sync_copy(zrows, zbuf)
            pltpu.sync_copy(zbuf, acc.at[pl.ds(s * BM, BM)])
            plsc.subcore_barrier()
            wrow = (r * NSUB + s) * NCH
            ya = (yq0, yq2)[ph]
            yb = (yq1, yq3)[ph]

            def group(g, carry):
                pltpu.sync_copy(rowg.at[pl.ds(wrow + g * GRP, GRP)], igb)
                pltpu.sync_copy(colr.at[pl.ds(wrow + g * GRP, GRP)], iscb)

                def gath(k):
                    b = k % NBUF

                    @pl.when(c == 0)
                    def _():
                        pltpu.async_copy(ya.at[igb.at[k]], rbufs[b], gsems[b])

                    @pl.when(c == 1)
                    def _():
                        pltpu.async_copy(yb.at[igb.at[k]], rbufs[b], gsems[b])

                def wait_scat(k):
                    b = k % NBUF
                    pltpu.make_async_copy(
                        rbufs[b], acc.at[iscb.at[k]], ssems[b]).wait()

                for k in range(LEAD):
                    gath(k)
                for k in range(GRP):
                    b = k % NBUF
                    if k + LEAD < GRP:
                        if k >= NBUF - LEAD:
                            wait_scat(k - (NBUF - LEAD))
                        gath(k + LEAD)
                    pltpu.make_async_copy(
                        ya.at[igb.at[k]], rbufs[b], gsems[b]).wait()
                    pltpu.async_copy(rbufs[b], acc.at[iscb.at[k]],
                                     ssems[b], add=True)
                for k in range(GRP - NBUF, GRP):
                    wait_scat(k)
                return carry

            lax.fori_loop(0, NG, group, 0)
            plsc.subcore_barrier()
            pltpu.sync_copy(acc.at[pl.ds(s * BM, BM)], zbuf)
            pltpu.sync_copy(zbuf, z_out.at[r, 2 * ph + c, pl.ds(s * BM, BM)])
            plsc.subcore_barrier()


_prop = functools.partial(
    pl.kernel,
    out_type=jax.ShapeDtypeStruct((R, 4, NP, 16), jnp.float32),
    mesh=_mesh,
    compiler_params=_sc_params,
    scratch_types=[
        pltpu.VMEM_SHARED((NP, 16), jnp.float32),
        pltpu.VMEM((GRP, 128), jnp.int32),
        pltpu.VMEM((GRP, 128), jnp.int32),
        pltpu.VMEM((128, 16), jnp.float32),
        pltpu.VMEM((128, 16), jnp.float32),
        pltpu.VMEM((128, 16), jnp.float32),
        pltpu.VMEM((128, 16), jnp.float32),
        pltpu.VMEM((BM, 16), jnp.float32),
        pltpu.SemaphoreType.DMA,
        pltpu.SemaphoreType.DMA,
        pltpu.SemaphoreType.DMA,
        pltpu.SemaphoreType.DMA,
        pltpu.SemaphoreType.DMA,
        pltpu.SemaphoreType.DMA,
        pltpu.SemaphoreType.DMA,
        pltpu.SemaphoreType.DMA,
    ],
)(_prop_body)


def _dense1_body(phi0, phi1, dis, diag, zq0, zq1, zq2, zq3,
                 x_o, y1q0_o, y1q1_o, y1q2_o, y1q3_o):
    z0 = jnp.concatenate([zq0[0], zq1[0], zq2[0], zq3[0]], axis=1)
    x = DT * phi1[0] + phi0[0] + C1 * (dis[0] * z0 + diag[0] * phi0[0])
    x_o[0] = x
    y1 = dis[0] * x
    for q, ref in enumerate((y1q0_o, y1q1_o, y1q2_o, y1q3_o)):
        ref[0] = y1[:, q * 16:(q + 1) * 16]


def _dense1(phi0, phi1, dis, diag, zq):
    f64 = jax.ShapeDtypeStruct((R, NP, HID), jnp.float32)
    f16o = jax.ShapeDtypeStruct((R, NP, 16), jnp.float32)
    bs64 = pl.BlockSpec((1, BM, HID), lambda r, i: (r, i, 0))
    bs16 = pl.BlockSpec((1, BM, 16), lambda r, i: (r, i, 0))
    bs1 = pl.BlockSpec((1, BM, 1), lambda r, i: (r, i, 0))
    return pl.pallas_call(
        _dense1_body,
        grid=(R, NSUB),
        in_specs=[bs64, bs64, bs1, bs1, bs16, bs16, bs16, bs16],
        out_specs=[bs64, bs16, bs16, bs16, bs16],
        out_shape=[f64, f16o, f16o, f16o, f16o],
    )(phi0, phi1, dis, diag, zq[:, 0], zq[:, 1], zq[:, 2], zq[:, 3])


def _dense2_body(x, phi0, dis, diag, zq0, zq1, zq2, zq3, y2_o):
    z1 = jnp.concatenate([zq0[0], zq1[0], zq2[0], zq3[0]], axis=1)
    y2_o[0] = (C2 * (dis[0] * z1 + diag[0] * x[0])
               + 2.0 * x[0] - phi0[0])


def _dense2(x, phi0, dis, diag, zq):
    f64 = jax.ShapeDtypeStruct((R, NP, HID), jnp.float32)
    bs64 = pl.BlockSpec((1, BM, HID), lambda r, i: (r, i, 0))
    bs16 = pl.BlockSpec((1, BM, 16), lambda r, i: (r, i, 0))
    bs1 = pl.BlockSpec((1, BM, 1), lambda r, i: (r, i, 0))
    return pl.pallas_call(
        _dense2_body,
        grid=(R, NSUB),
        in_specs=[bs64, bs64, bs1, bs1, bs16, bs16, bs16, bs16],
        out_specs=[bs64],
        out_shape=[f64],
    )(x, phi0, dis, diag, zq[:, 0], zq[:, 1], zq[:, 2], zq[:, 3])[0]


def _bg_body(y2f, bn, og, idx0, idx1, rows, sem):
    c = lax.axis_index("c")
    s = lax.axis_index("s")
    w = s * 2 + c
    pltpu.sync_copy(bn.at[pl.ds(w * 128, 128)], idx0)
    for q in range(8):
        idx1[pl.ds(q * 16, 16)] = idx0[pl.ds(q * 16, 16)] + NP
    pltpu.async_copy(y2f.at[idx0], rows, sem).wait()
    pltpu.sync_copy(rows, og.at[0, pl.ds(w * 128, 128)])
    pltpu.async_copy(y2f.at[idx1], rows, sem).wait()
    pltpu.sync_copy(rows, og.at[1, pl.ds(w * 128, 128)])


_bg = functools.partial(
    pl.kernel,
    out_type=jax.ShapeDtypeStruct((R, B, HID), jnp.float32),
    mesh=_mesh,
    compiler_params=_sc_params,
    scratch_types=[
        pltpu.VMEM((128,), jnp.int32),
        pltpu.VMEM((128,), jnp.int32),
        pltpu.VMEM((128, HID), jnp.float32),
        pltpu.SemaphoreType.DMA,
    ],
)(_bg_body)


def _chunkify(a, padval):
    a = a.reshape(R, NSUB, 50000)
    a = jnp.pad(a, ((0, 0), (0, 0), (0, 48)), constant_values=padval)
    return a.reshape(R * NSUB * NCH, 128)


def kernel(features, multi_r_data, batch_nodes, W0, b0, W1, b1):
    featp = jnp.pad(features, ((0, NP - N), (0, 0)))
    mr = multi_r_data.reshape(R, 2, 6250, 128)
    colr, rows_, rowg = _prep(mr)
    colr_c = _chunkify(colr, TRASH)
    rows_c = _chunkify(rows_, TRASH)
    rowg_c = _chunkify(rowg, 0)

    zeros1 = jnp.zeros((BM,), jnp.float32)
    zrows = jnp.zeros((BM, 16), jnp.float32)

    degf, selff = _deg(colr_c, rows_c, zeros1)
    phi0, phi1, y0q0, y0q1, y0q2, y0q3, dis, diag = _mm(
        featp, W0, b0.reshape(R, 1, HID), W1, b1.reshape(R, 1, HID),
        degf.reshape(R, NP, 1), selff.reshape(R, NP, 1))

    z0 = _prop(y0q0.reshape(R * NP, 16), y0q1.reshape(R * NP, 16),
               y0q2.reshape(R * NP, 16), y0q3.reshape(R * NP, 16),
               rowg_c, colr_c, zrows)
    x, y1q0, y1q1, y1q2, y1q3 = _dense1(phi0, phi1, dis, diag, z0)
    z1 = _prop(y1q0.reshape(R * NP, 16), y1q1.reshape(R * NP, 16),
               y1q2.reshape(R * NP, 16), y1q3.reshape(R * NP, 16),
               rowg_c, colr_c, zrows)
    y2 = _dense2(x, phi0, dis, diag, z1)

    og = _bg(y2.reshape(R * NP, HID), batch_nodes)
    return jnp.transpose(og, (1, 0, 2)).reshape(B, R * HID)


# speedup vs baseline: 16.0257x; 1.3445x over previous
import functools

import jax
import jax.numpy as jnp
from jax import lax
from jax.experimental import pallas as pl
from jax.experimental.pallas import tpu as pltpu
from jax.experimental.pallas import tpu_sc as plsc

N = 50000
NP = 50048
E = 800000
NCH = 391
TRASH = 50000
HID = 64
DT = 0.5
C1 = DT * DT / 2.0
C2 = DT * DT
NSUB = 16
BM = 3128
R = 2
B = 4096
K = 256
NC2 = 196
ESUB = NC2 * K
GRP = 7
NG = 28
NBUF = 4
LEAD = 2

_mesh = plsc.VectorSubcoreMesh(core_axis_name="c", subcore_axis_name="s")
_sc_params = pltpu.CompilerParams(use_tc_tiling_on_sc=False)


def _prep_body(mr_ref, colr_ref, cold_ref, rowg_ref):
    r = pl.program_id(0)
    row = mr_ref[0, 0]
    col = mr_ref[0, 1]
    self_m = row == col
    colr_ref[0] = jnp.where(self_m, TRASH, col)
    cold_ref[0] = jnp.where(self_m, NP + col, col)
    rowg_ref[0] = row + r * NP


def _prep(mr):
    out = jax.ShapeDtypeStruct((R, 6250, 128), jnp.int32)
    return pl.pallas_call(
        _prep_body,
        grid=(R,),
        in_specs=[pl.BlockSpec((1, 2, 6250, 128), lambda r: (r, 0, 0, 0))],
        out_specs=[pl.BlockSpec((1, 6250, 128), lambda r: (r, 0, 0))] * 3,
        out_shape=[out, out, out],
    )(mr)


BM2 = 2 * NP // NSUB


def _deg_body(cold, zer, deg_out, dacc, ones_v, icb, dbuf, dsem):
    c = lax.axis_index("c")
    s = lax.axis_index("s")
    for q in range(K // 16):
        ones_v[pl.ds(q * 16, 16)] = jnp.full((16,), 1.0, jnp.float32)
    pltpu.sync_copy(zer, dbuf)
    pltpu.sync_copy(dbuf, dacc.at[pl.ds(s * BM2, BM2)])
    plsc.subcore_barrier()
    wrow = (c * NSUB + s) * NC2

    def group(g, carry):
        pltpu.sync_copy(cold.at[pl.ds(wrow + g * GRP, GRP)], icb)
        for k in range(GRP):
            pltpu.async_copy(ones_v, dacc.at[icb.at[k]], dsem, add=True)
        for k in range(GRP):
            pltpu.make_async_copy(ones_v, dacc.at[icb.at[k]], dsem).wait()
        return carry

    lax.fori_loop(0, NG, group, 0)
    plsc.subcore_barrier()
    pltpu.sync_copy(dacc.at[pl.ds(s * BM2, BM2)], dbuf)
    pltpu.sync_copy(dbuf, deg_out.at[pl.ds(c * 2 * NP + s * BM2, BM2)])


_deg = functools.partial(
    pl.kernel,
    out_type=jax.ShapeDtypeStruct((R * 2 * NP,), jnp.float32),
    mesh=_mesh,
    compiler_params=_sc_params,
    scratch_types=[
        pltpu.VMEM_SHARED((2 * NP,), jnp.float32),
        pltpu.VMEM((K,), jnp.float32),
        pltpu.VMEM((GRP, K), jnp.int32),
        pltpu.VMEM((BM2,), jnp.float32),
        pltpu.SemaphoreType.DMA,
    ],
)(_deg_body)


def _mm_body(feat, w0, bb0, w1, bb1, deg, selfc,
             phi0_o, phi1_o, yq0_o, yq1_o, yq2_o, yq3_o, dis_o, diag_o):
    f = feat[...]
    p0 = jnp.dot(f, w0[0], preferred_element_type=jnp.float32) + bb0[0]
    p1 = jnp.dot(f, w1[0], preferred_element_type=jnp.float32) + bb1[0]
    loop = jnp.where(selfc[0] > 0.0, 1.0, 0.0)
    dg = deg[0] + loop
    pos = dg > 0.0
    safe = jnp.maximum(dg, 1.0)
    dis = jnp.where(pos, lax.rsqrt(safe), 0.0)
    diag = jnp.where(pos, loop / safe, 0.0)
    phi0_o[0] = p0
    phi1_o[0] = p1
    y0 = p0 * dis
    for q, ref in enumerate((yq0_o, yq1_o, yq2_o, yq3_o)):
        ref[0] = y0[:, q * 16:(q + 1) * 16]
    dis_o[0] = dis
    diag_o[0] = diag


def _mm(featp, W0, b0, W1, b1, deg, selfc):
    f64 = jax.ShapeDtypeStruct((R, NP, HID), jnp.float32)
    f16o = jax.ShapeDtypeStruct((R, NP, 16), jnp.float32)
    f1 = jax.ShapeDtypeStruct((R, NP, 1), jnp.float32)
    bs16 = pl.BlockSpec((1, BM, 16), lambda r, i: (r, i, 0))
    return pl.pallas_call(
        _mm_body,
        grid=(R, NSUB),
        in_specs=[
            pl.BlockSpec((BM, 128), lambda r, i: (i, 0)),
            pl.BlockSpec((1, 128, HID), lambda r, i: (r, 0, 0)),
            pl.BlockSpec((1, 1, HID), lambda r, i: (r, 0, 0)),
            pl.BlockSpec((1, 128, HID), lambda r, i: (r, 0, 0)),
            pl.BlockSpec((1, 1, HID), lambda r, i: (r, 0, 0)),
            pl.BlockSpec((1, BM, 1), lambda r, i: (r, i, 0)),
            pl.BlockSpec((1, BM, 1), lambda r, i: (r, i, 0)),
        ],
        out_specs=[
            pl.BlockSpec((1, BM, HID), lambda r, i: (r, i, 0)),
            pl.BlockSpec((1, BM, HID), lambda r, i: (r, i, 0)),
            bs16, bs16, bs16, bs16,
            pl.BlockSpec((1, BM, 1), lambda r, i: (r, i, 0)),
            pl.BlockSpec((1, BM, 1), lambda r, i: (r, i, 0)),
        ],
        out_shape=[f64, f64, f16o, f16o, f16o, f16o, f1, f1],
    )(featp, W0, b0, W1, b1, deg, selfc)


def _prop_body(yq0, yq1, yq2, yq3, rowg, colr, zrows, z_out,
               acc, igb, iscb, rb0, rb1, rb2, rb3, zbuf,
               gs0, gs1, gs2, gs3, ss0, ss1, ss2, ss3):
    c = lax.axis_index("c")
    s = lax.axis_index("s")
    rbufs = (rb0, rb1, rb2, rb3)
    gsems = (gs0, gs1, gs2, gs3)
    ssems = (ss0, ss1, ss2, ss3)
    for r in range(R):
        for ph in range(2):
            pltpu.sync_copy(zrows, zbuf)
            pltpu.sync_copy(zbuf, acc.at[pl.ds(s * BM, BM)])
            plsc.subcore_barrier()
            wrow = (r * NSUB + s) * NC2
            ya = (yq0, yq2)[ph]
            yb = (yq1, yq3)[ph]

            def group(g, carry):
                pltpu.sync_copy(rowg.at[pl.ds(wrow + g * GRP, GRP)], igb)
                pltpu.sync_copy(colr.at[pl.ds(wrow + g * GRP, GRP)], iscb)

                def gath(k):
                    b = k % NBUF

                    @pl.when(c == 0)
                    def _():
                        pltpu.async_copy(ya.at[igb.at[k]], rbufs[b], gsems[b])

                    @pl.when(c == 1)
                    def _():
                        pltpu.async_copy(yb.at[igb.at[k]], rbufs[b], gsems[b])

                def wait_scat(k):
                    b = k % NBUF
                    pltpu.make_async_copy(
                        rbufs[b], acc.at[iscb.at[k]], ssems[b]).wait()

                for k in range(LEAD):
                    gath(k)
                for k in range(GRP):
                    b = k % NBUF
                    if k + LEAD < GRP:
                        if k >= NBUF - LEAD:
                            wait_scat(k - (NBUF - LEAD))
                        gath(k + LEAD)
                    pltpu.make_async_copy(
                        ya.at[igb.at[k]], rbufs[b], gsems[b]).wait()
                    pltpu.async_copy(rbufs[b], acc.at[iscb.at[k]],
                                     ssems[b], add=True)
                for k in range(GRP - NBUF, GRP):
                    wait_scat(k)
                return carry

            lax.fori_loop(0, NG, group, 0)
            plsc.subcore_barrier()
            pltpu.sync_copy(acc.at[pl.ds(s * BM, BM)], zbuf)
            pltpu.sync_copy(zbuf, z_out.at[r, 2 * ph + c, pl.ds(s * BM, BM)])
            plsc.subcore_barrier()


_prop = functools.partial(
    pl.kernel,
    out_type=jax.ShapeDtypeStruct((R, 4, NP, 16), jnp.float32),
    mesh=_mesh,
    compiler_params=_sc_params,
    scratch_types=[
        pltpu.VMEM_SHARED((NP, 16), jnp.float32),
        pltpu.VMEM((GRP, K), jnp.int32),
        pltpu.VMEM((GRP, K), jnp.int32),
        pltpu.VMEM((K, 16), jnp.float32),
        pltpu.VMEM((K, 16), jnp.float32),
        pltpu.VMEM((K, 16), jnp.float32),
        pltpu.VMEM((K, 16), jnp.float32),
        pltpu.VMEM((BM, 16), jnp.float32),
        pltpu.SemaphoreType.DMA,
        pltpu.SemaphoreType.DMA,
        pltpu.SemaphoreType.DMA,
        pltpu.SemaphoreType.DMA,
        pltpu.SemaphoreType.DMA,
        pltpu.SemaphoreType.DMA,
        pltpu.SemaphoreType.DMA,
        pltpu.SemaphoreType.DMA,
    ],
)(_prop_body)


def _dense1_body(phi0, phi1, dis, diag, zq0, zq1, zq2, zq3,
                 x_o, y1q0_o, y1q1_o, y1q2_o, y1q3_o):
    z0 = jnp.concatenate([zq0[0], zq1[0], zq2[0], zq3[0]], axis=1)
    x = DT * phi1[0] + phi0[0] + C1 * (dis[0] * z0 + diag[0] * phi0[0])
    x_o[0] = x
    y1 = dis[0] * x
    for q, ref in enumerate((y1q0_o, y1q1_o, y1q2_o, y1q3_o)):
        ref[0] = y1[:, q * 16:(q + 1) * 16]


def _dense1(phi0, phi1, dis, diag, zq):
    f64 = jax.ShapeDtypeStruct((R, NP, HID), jnp.float32)
    f16o = jax.ShapeDtypeStruct((R, NP, 16), jnp.float32)
    bs64 = pl.BlockSpec((1, BM, HID), lambda r, i: (r, i, 0))
    bs16 = pl.BlockSpec((1, BM, 16), lambda r, i: (r, i, 0))
    bs1 = pl.BlockSpec((1, BM, 1), lambda r, i: (r, i, 0))
    return pl.pallas_call(
        _dense1_body,
        grid=(R, NSUB),
        in_specs=[bs64, bs64, bs1, bs1, bs16, bs16, bs16, bs16],
        out_specs=[bs64, bs16, bs16, bs16, bs16],
        out_shape=[f64, f16o, f16o, f16o, f16o],
    )(phi0, phi1, dis, diag, zq[:, 0], zq[:, 1], zq[:, 2], zq[:, 3])


def _dense2_body(x, phi0, dis, diag, zq0, zq1, zq2, zq3, y2_o):
    z1 = jnp.concatenate([zq0[0], zq1[0], zq2[0], zq3[0]], axis=1)
    y2_o[0] = (C2 * (dis[0] * z1 + diag[0] * x[0])
               + 2.0 * x[0] - phi0[0])


def _dense2(x, phi0, dis, diag, zq):
    f64 = jax.ShapeDtypeStruct((R, NP, HID), jnp.float32)
    bs64 = pl.BlockSpec((1, BM, HID), lambda r, i: (r, i, 0))
    bs16 = pl.BlockSpec((1, BM, 16), lambda r, i: (r, i, 0))
    bs1 = pl.BlockSpec((1, BM, 1), lambda r, i: (r, i, 0))
    return pl.pallas_call(
        _dense2_body,
        grid=(R, NSUB),
        in_specs=[bs64, bs64, bs1, bs1, bs16, bs16, bs16, bs16],
        out_specs=[bs64],
        out_shape=[f64],
    )(x, phi0, dis, diag, zq[:, 0], zq[:, 1], zq[:, 2], zq[:, 3])[0]


def _bg_body(y2f, bn, og, idx0, idx1, rows, sem):
    c = lax.axis_index("c")
    s = lax.axis_index("s")
    w = s * 2 + c
    pltpu.sync_copy(bn.at[pl.ds(w * 128, 128)], idx0)
    for q in range(8):
        idx1[pl.ds(q * 16, 16)] = idx0[pl.ds(q * 16, 16)] + NP
    pltpu.async_copy(y2f.at[idx0], rows, sem).wait()
    pltpu.sync_copy(rows, og.at[0, pl.ds(w * 128, 128)])
    pltpu.async_copy(y2f.at[idx1], rows, sem).wait()
    pltpu.sync_copy(rows, og.at[1, pl.ds(w * 128, 128)])


_bg = functools.partial(
    pl.kernel,
    out_type=jax.ShapeDtypeStruct((R, B, HID), jnp.float32),
    mesh=_mesh,
    compiler_params=_sc_params,
    scratch_types=[
        pltpu.VMEM((128,), jnp.int32),
        pltpu.VMEM((128,), jnp.int32),
        pltpu.VMEM((128, HID), jnp.float32),
        pltpu.SemaphoreType.DMA,
    ],
)(_bg_body)


def _chunkify(a, padval):
    a = a.reshape(R, NSUB, 50000)
    a = jnp.pad(a, ((0, 0), (0, 0), (0, ESUB - 50000)), constant_values=padval)
    return a.reshape(R * NSUB * NC2, K)


def kernel(features, multi_r_data, batch_nodes, W0, b0, W1, b1):
    featp = jnp.pad(features, ((0, NP - N), (0, 0)))
    mr = multi_r_data.reshape(R, 2, 6250, 128)
    colr, cold, rowg = _prep(mr)
    colr_c = _chunkify(colr, TRASH)
    cold_c = _chunkify(cold, TRASH)
    rowg_c = _chunkify(rowg, 0)

    zeros2 = jnp.zeros((BM2,), jnp.float32)
    zrows = jnp.zeros((BM, 16), jnp.float32)

    dv = _deg(cold_c, zeros2).reshape(R, 2, NP)
    phi0, phi1, y0q0, y0q1, y0q2, y0q3, dis, diag = _mm(
        featp, W0, b0.reshape(R, 1, HID), W1, b1.reshape(R, 1, HID),
        dv[:, 0].reshape(R, NP, 1), dv[:, 1].reshape(R, NP, 1))

    z0 = _prop(y0q0.reshape(R * NP, 16), y0q1.reshape(R * NP, 16),
               y0q2.reshape(R * NP, 16), y0q3.reshape(R * NP, 16),
               rowg_c, colr_c, zrows)
    x, y1q0, y1q1, y1q2, y1q3 = _dense1(phi0, phi1, dis, diag, z0)
    z1 = _prop(y1q0.reshape(R * NP, 16), y1q1.reshape(R * NP, 16),
               y1q2.reshape(R * NP, 16), y1q3.reshape(R * NP, 16),
               rowg_c, colr_c, zrows)
    y2 = _dense2(x, phi0, dis, diag, z1)

    og = _bg(y2.reshape(R * NP, HID), batch_nodes)
    return jnp.transpose(og, (1, 0, 2)).reshape(B, R * HID)


# speedup vs baseline: 17.5766x; 1.0968x over previous
import functools

import jax
import jax.numpy as jnp
from jax import lax
from jax.experimental import pallas as pl
from jax.experimental.pallas import tpu as pltpu
from jax.experimental.pallas import tpu_sc as plsc

N = 50000
NP = 50048
E = 800000
NCH = 391
TRASH = 50000
HID = 64
DT = 0.5
C1 = DT * DT / 2.0
C2 = DT * DT
NSUB = 16
BM = 3128
R = 2
B = 4096
K = 256
NC2 = 196
ESUB = NC2 * K
GRP = 7
NG = 28
NBUF = 4
LEAD = 2

_mesh = plsc.VectorSubcoreMesh(core_axis_name="c", subcore_axis_name="s")
_sc_params = pltpu.CompilerParams(use_tc_tiling_on_sc=False)


def _prep_body(mr_ref, colr_ref, cold_ref, rowg_ref):
    r = pl.program_id(0)
    row = mr_ref[0, 0]
    col = mr_ref[0, 1]
    self_m = row == col
    colr_ref[0] = jnp.where(self_m, TRASH, col)
    cold_ref[0] = jnp.where(self_m, NP + col, col)
    rowg_ref[0] = row + r * NP


def _prep(mr):
    out = jax.ShapeDtypeStruct((R, 6250, 128), jnp.int32)
    return pl.pallas_call(
        _prep_body,
        grid=(R,),
        in_specs=[pl.BlockSpec((1, 2, 6250, 128), lambda r: (r, 0, 0, 0))],
        out_specs=[pl.BlockSpec((1, 6250, 128), lambda r: (r, 0, 0))] * 3,
        out_shape=[out, out, out],
    )(mr)


BM2 = 2 * NP // NSUB


def _deg_body(cold, zer, deg_out, dacc, ones_v, icb, dbuf, dsem):
    c = lax.axis_index("c")
    s = lax.axis_index("s")
    for q in range(K // 16):
        ones_v[pl.ds(q * 16, 16)] = jnp.full((16,), 1.0, jnp.float32)
    pltpu.sync_copy(zer, dbuf)
    pltpu.sync_copy(dbuf, dacc.at[pl.ds(s * BM2, BM2)])
    plsc.subcore_barrier()
    wrow = (c * NSUB + s) * NC2

    def group(g, carry):
        pltpu.sync_copy(cold.at[pl.ds(wrow + g * GRP, GRP)], icb)
        for k in range(GRP):
            pltpu.async_copy(ones_v, dacc.at[icb.at[k]], dsem, add=True)
        for k in range(GRP):
            pltpu.make_async_copy(ones_v, dacc.at[icb.at[k]], dsem).wait()
        return carry

    lax.fori_loop(0, NG, group, 0)
    plsc.subcore_barrier()
    pltpu.sync_copy(dacc.at[pl.ds(s * BM2, BM2)], dbuf)
    pltpu.sync_copy(dbuf, deg_out.at[pl.ds(c * 2 * NP + s * BM2, BM2)])


_deg = functools.partial(
    pl.kernel,
    out_type=jax.ShapeDtypeStruct((R * 2 * NP,), jnp.float32),
    mesh=_mesh,
    compiler_params=_sc_params,
    scratch_types=[
        pltpu.VMEM_SHARED((2 * NP,), jnp.float32),
        pltpu.VMEM((K,), jnp.float32),
        pltpu.VMEM((GRP, K), jnp.int32),
        pltpu.VMEM((BM2,), jnp.float32),
        pltpu.SemaphoreType.DMA,
    ],
)(_deg_body)


def _mm_body(feat, w0, bb0, w1, bb1, deg, selfc,
             phi0_o, phi1_o, yq0_o, yq1_o, yq2_o, yq3_o, dis_o, diag_o):
    f = feat[...]
    p0 = jnp.dot(f, w0[0], preferred_element_type=jnp.float32) + bb0[0]
    p1 = jnp.dot(f, w1[0], preferred_element_type=jnp.float32) + bb1[0]
    loop = jnp.where(selfc[0] > 0.0, 1.0, 0.0)
    dg = deg[0] + loop
    pos = dg > 0.0
    safe = jnp.maximum(dg, 1.0)
    dis = jnp.where(pos, lax.rsqrt(safe), 0.0)
    diag = jnp.where(pos, loop / safe, 0.0)
    phi0_o[0] = p0
    phi1_o[0] = p1
    y0 = p0 * dis
    for q, ref in enumerate((yq0_o, yq1_o, yq2_o, yq3_o)):
        ref[0] = y0[:, q * 16:(q + 1) * 16]
    dis_o[0] = dis
    diag_o[0] = diag


def _mm(featp, W0, b0, W1, b1, deg, selfc):
    f64 = jax.ShapeDtypeStruct((R, NP, HID), jnp.float32)
    f16o = jax.ShapeDtypeStruct((R, NP, 16), jnp.float32)
    f1 = jax.ShapeDtypeStruct((R, NP, 1), jnp.float32)
    bs16 = pl.BlockSpec((1, BM, 16), lambda r, i: (r, i, 0))
    return pl.pallas_call(
        _mm_body,
        grid=(R, NSUB),
        in_specs=[
            pl.BlockSpec((BM, 128), lambda r, i: (i, 0)),
            pl.BlockSpec((1, 128, HID), lambda r, i: (r, 0, 0)),
            pl.BlockSpec((1, 1, HID), lambda r, i: (r, 0, 0)),
            pl.BlockSpec((1, 128, HID), lambda r, i: (r, 0, 0)),
            pl.BlockSpec((1, 1, HID), lambda r, i: (r, 0, 0)),
            pl.BlockSpec((1, BM, 1), lambda r, i: (r, i, 0)),
            pl.BlockSpec((1, BM, 1), lambda r, i: (r, i, 0)),
        ],
        out_specs=[
            pl.BlockSpec((1, BM, HID), lambda r, i: (r, i, 0)),
            pl.BlockSpec((1, BM, HID), lambda r, i: (r, i, 0)),
            bs16, bs16, bs16, bs16,
            pl.BlockSpec((1, BM, 1), lambda r, i: (r, i, 0)),
            pl.BlockSpec((1, BM, 1), lambda r, i: (r, i, 0)),
        ],
        out_shape=[f64, f64, f16o, f16o, f16o, f16o, f1, f1],
    )(featp, W0, b0, W1, b1, deg, selfc)


def _run_phases(c, s, yqs, rowg, colr, zrows, z_out, acc, igb, iscb,
                rbufs, zbuf, gsems, ssems):
    for r in range(R):
        for ph in range(2):
            pltpu.sync_copy(zrows, zbuf)
            pltpu.sync_copy(zbuf, acc.at[pl.ds(s * BM, 1568)])
            pltpu.sync_copy(zbuf.at[pl.ds(0, 1560)],
                            acc.at[pl.ds(s * BM + 1568, 1560)])
            plsc.subcore_barrier()
            wrow = (r * NSUB + s) * NC2
            ya = (yqs[0], yqs[2])[ph]
            yb = (yqs[1], yqs[3])[ph]

            def group(g, carry):
                pltpu.sync_copy(rowg.at[pl.ds(wrow + g * GRP, GRP)], igb)
                pltpu.sync_copy(colr.at[pl.ds(wrow + g * GRP, GRP)], iscb)

                def gath(k):
                    b = k % NBUF

                    @pl.when(c == 0)
                    def _():
                        pltpu.async_copy(ya.at[igb.at[k]], rbufs[b], gsems[b])

                    @pl.when(c == 1)
                    def _():
                        pltpu.async_copy(yb.at[igb.at[k]], rbufs[b], gsems[b])

                def wait_scat(k):
                    b = k % NBUF
                    pltpu.make_async_copy(
                        rbufs[b], acc.at[iscb.at[k]], ssems[b]).wait()

                for k in range(LEAD):
                    gath(k)
                for k in range(GRP):
                    b = k % NBUF
                    if k + LEAD < GRP:
                        if k >= NBUF - LEAD:
                            wait_scat(k - (NBUF - LEAD))
                        gath(k + LEAD)
                    pltpu.make_async_copy(
                        ya.at[igb.at[k]], rbufs[b], gsems[b]).wait()
                    pltpu.async_copy(rbufs[b], acc.at[iscb.at[k]],
                                     ssems[b], add=True)
                for k in range(GRP - NBUF, GRP):
                    wait_scat(k)
                return carry

            lax.fori_loop(0, NG, group, 0)
            plsc.subcore_barrier()
            qq = 2 * ph + c
            zbase = (r * 4 + qq) * NP + s * BM
            pltpu.sync_copy(acc.at[pl.ds(s * BM, 1568)], zbuf)
            pltpu.sync_copy(zbuf, z_out.at[pl.ds(zbase, 1568)])
            pltpu.sync_copy(acc.at[pl.ds(s * BM + 1568, 1560)],
                            zbuf.at[pl.ds(0, 1560)])
            pltpu.sync_copy(zbuf.at[pl.ds(0, 1560)],
                            z_out.at[pl.ds(zbase + 1568, 1560)])
            plsc.subcore_barrier()


_PROP_SCRATCH = [
    pltpu.VMEM_SHARED((NP, 16), jnp.float32),
    pltpu.VMEM((GRP, K), jnp.int32),
    pltpu.VMEM((GRP, K), jnp.int32),
    pltpu.VMEM((K, 16), jnp.float32),
    pltpu.VMEM((K, 16), jnp.float32),
    pltpu.VMEM((K, 16), jnp.float32),
    pltpu.VMEM((K, 16), jnp.float32),
    pltpu.VMEM((1568, 16), jnp.float32),
    pltpu.SemaphoreType.DMA,
    pltpu.SemaphoreType.DMA,
    pltpu.SemaphoreType.DMA,
    pltpu.SemaphoreType.DMA,
    pltpu.SemaphoreType.DMA,
    pltpu.SemaphoreType.DMA,
    pltpu.SemaphoreType.DMA,
    pltpu.SemaphoreType.DMA,
]


def _prop_body(yq0, yq1, yq2, yq3, rowg, colr, zrows, z_out,
               acc, igb, iscb, rb0, rb1, rb2, rb3, zbuf,
               gs0, gs1, gs2, gs3, ss0, ss1, ss2, ss3):
    c = lax.axis_index("c")
    s = lax.axis_index("s")
    _run_phases(c, s, (yq0, yq1, yq2, yq3), rowg, colr, zrows, z_out,
                acc, igb, iscb, (rb0, rb1, rb2, rb3), zbuf,
                (gs0, gs1, gs2, gs3), (ss0, ss1, ss2, ss3))


_prop = functools.partial(
    pl.kernel,
    out_type=jax.ShapeDtypeStruct((R * 4 * NP, 16), jnp.float32),
    mesh=_mesh,
    compiler_params=_sc_params,
    scratch_types=list(_PROP_SCRATCH),
)(_prop_body)


def _prop2f_body(yq0, yq1, yq2, yq3, rowg, colr, zrows,
                 uq0, uq1, uq2, uq3, disf, bn, z_out, og,
                 acc, igb, iscb, rb0, rb1, rb2, rb3, zbuf,
                 bnb, xib, zib0, zib1, zg0, zg1, ug0, ug1, ag, ot0, ot1,
                 gs0, gs1, gs2, gs3, ss0, ss1, ss2, ss3):
    c = lax.axis_index("c")
    s = lax.axis_index("s")
    _run_phases(c, s, (yq0, yq1, yq2, yq3), rowg, colr, zrows, z_out,
                acc, igb, iscb, (rb0, rb1, rb2, rb3), zbuf,
                (gs0, gs1, gs2, gs3), (ss0, ss1, ss2, ss3))

    def tail(uA, uB, qA, qB):
        for r in range(R):
            for cb in range(2):
                base = (s * 2 + cb) * 128
                pltpu.sync_copy(bn.at[pl.ds(base, 128)], bnb)
                for t in range(8):
                    sl = pl.ds(t * 16, 16)
                    bv = bnb[sl]
                    xib[sl] = bv + r * NP
                    zib0[sl] = bv + (r * 4 + qA) * NP
                    zib1[sl] = bv + (r * 4 + qB) * NP
                d0 = pltpu.async_copy(z_out.at[zib0], zg0, gs0)
                d1 = pltpu.async_copy(z_out.at[zib1], zg1, gs1)
                d2 = pltpu.async_copy(uA.at[xib], ug0, gs2)
                d3 = pltpu.async_copy(uB.at[xib], ug1, gs3)
                d4 = pltpu.async_copy(disf.at[xib], ag, ss0)
                d0.wait()
                d1.wait()
                d2.wait()
                d3.wait()
                d4.wait()

                def rowblk(t, carry):
                    av = C2 * ag[pl.ds(t * 16, 16)]
                    for j in range(16):
                        i = t * 16 + j
                        aj = jnp.broadcast_to(av[j], (16,))
                        ot0[i, :] = aj * zg0[i, :] + ug0[i, :]
                        ot1[i, :] = aj * zg1[i, :] + ug1[i, :]
                    return carry

                lax.fori_loop(0, 8, rowblk, 0)
                pltpu.sync_copy(ot0, og.at[pl.ds((r * 4 + qA) * B + base, 128)])
                pltpu.sync_copy(ot1, og.at[pl.ds((r * 4 + qB) * B + base, 128)])

    @pl.when(c == 0)
    def _():
        tail(uq0, uq2, 0, 2)

    @pl.when(c == 1)
    def _():
        tail(uq1, uq3, 1, 3)


_prop2f = functools.partial(
    pl.kernel,
    out_type=[jax.ShapeDtypeStruct((R * 4 * NP, 16), jnp.float32),
              jax.ShapeDtypeStruct((R * 4 * B, 16), jnp.float32)],
    mesh=_mesh,
    compiler_params=_sc_params,
    scratch_types=list(_PROP_SCRATCH[:8]) + [
        pltpu.VMEM((128,), jnp.int32),
        pltpu.VMEM((128,), jnp.int32),
        pltpu.VMEM((128,), jnp.int32),
        pltpu.VMEM((128,), jnp.int32),
        pltpu.VMEM((128, 16), jnp.float32),
        pltpu.VMEM((128, 16), jnp.float32),
        pltpu.VMEM((128, 16), jnp.float32),
        pltpu.VMEM((128, 16), jnp.float32),
        pltpu.VMEM((128,), jnp.float32),
        pltpu.VMEM((128, 16), jnp.float32),
        pltpu.VMEM((128, 16), jnp.float32),
    ] + list(_PROP_SCRATCH[8:]),
)(_prop2f_body)


def _dense1_body(phi0, phi1, dis, diag, zq0, zq1, zq2, zq3,
                 y1q0_o, y1q1_o, y1q2_o, y1q3_o,
                 uq0_o, uq1_o, uq2_o, uq3_o):
    z0 = jnp.concatenate([zq0[0], zq1[0], zq2[0], zq3[0]], axis=1)
    x = DT * phi1[0] + phi0[0] + C1 * (dis[0] * z0 + diag[0] * phi0[0])
    y1 = dis[0] * x
    u = (2.0 + C2 * diag[0]) * x - phi0[0]
    for q, ref in enumerate((y1q0_o, y1q1_o, y1q2_o, y1q3_o)):
        ref[0] = y1[:, q * 16:(q + 1) * 16]
    for q, ref in enumerate((uq0_o, uq1_o, uq2_o, uq3_o)):
        ref[0] = u[:, q * 16:(q + 1) * 16]


def _dense1(phi0, phi1, dis, diag, zq):
    f16o = jax.ShapeDtypeStruct((R, NP, 16), jnp.float32)
    bs64 = pl.BlockSpec((1, BM, HID), lambda r, i: (r, i, 0))
    bs16 = pl.BlockSpec((1, BM, 16), lambda r, i: (r, i, 0))
    bs1 = pl.BlockSpec((1, BM, 1), lambda r, i: (r, i, 0))
    return pl.pallas_call(
        _dense1_body,
        grid=(R, NSUB),
        in_specs=[bs64, bs64, bs1, bs1, bs16, bs16, bs16, bs16],
        out_specs=[bs16] * 8,
        out_shape=[f16o] * 8,
    )(phi0, phi1, dis, diag, zq[:, 0], zq[:, 1], zq[:, 2], zq[:, 3])


def _chunkify(a, padval):
    a = a.reshape(R, NSUB, 50000)
    a = jnp.pad(a, ((0, 0), (0, 0), (0, ESUB - 50000)), constant_values=padval)
    return a.reshape(R * NSUB * NC2, K)


def kernel(features, multi_r_data, batch_nodes, W0, b0, W1, b1):
    featp = jnp.pad(features, ((0, NP - N), (0, 0)))
    mr = multi_r_data.reshape(R, 2, 6250, 128)
    colr, cold, rowg = _prep(mr)
    colr_c = _chunkify(colr, TRASH)
    cold_c = _chunkify(cold, TRASH)
    rowg_c = _chunkify(rowg, 0)

    zeros2 = jnp.zeros((BM2,), jnp.float32)
    zrows = jnp.zeros((1568, 16), jnp.float32)

    dv = _deg(cold_c, zeros2).reshape(R, 2, NP)
    phi0, phi1, y0q0, y0q1, y0q2, y0q3, dis, diag = _mm(
        featp, W0, b0.reshape(R, 1, HID), W1, b1.reshape(R, 1, HID),
        dv[:, 0].reshape(R, NP, 1), dv[:, 1].reshape(R, NP, 1))

    z0 = _prop(y0q0.reshape(R * NP, 16), y0q1.reshape(R * NP, 16),
               y0q2.reshape(R * NP, 16), y0q3.reshape(R * NP, 16),
               rowg_c, colr_c, zrows).reshape(R, 4, NP, 16)
    y1q0, y1q1, y1q2, y1q3, uq0, uq1, uq2, uq3 = _dense1(
        phi0, phi1, dis, diag, z0)
    _, og = _prop2f(y1q0.reshape(R * NP, 16), y1q1.reshape(R * NP, 16),
                    y1q2.reshape(R * NP, 16), y1q3.reshape(R * NP, 16),
                    rowg_c, colr_c, zrows,
                    uq0.reshape(R * NP, 16), uq1.reshape(R * NP, 16),
                    uq2.reshape(R * NP, 16), uq3.reshape(R * NP, 16),
                    dis.reshape(R * NP), batch_nodes)
    og = og.reshape(R, 4, B, 16)
    return jnp.transpose(og, (2, 0, 1, 3)).reshape(B, R * HID)


# speedup vs baseline: 19.9328x; 1.1340x over previous
import functools

import jax
import jax.numpy as jnp
from jax import lax
from jax.experimental import pallas as pl
from jax.experimental.pallas import tpu as pltpu
from jax.experimental.pallas import tpu_sc as plsc

N = 50000
NP = 50048
E = 800000
NCH = 391
TRASH = 50000
HID = 64
DT = 0.5
C1 = DT * DT / 2.0
C2 = DT * DT
NSUB = 16
BM = 3128
R = 2
B = 4096
K = 256
NC2 = 196
ESUB = NC2 * K
GRP = 14
NG = 14
NBUF = 4
LEAD = 2

_mesh = plsc.VectorSubcoreMesh(core_axis_name="c", subcore_axis_name="s")
_sc_params = pltpu.CompilerParams(use_tc_tiling_on_sc=False)


def _prep_body(mr_ref, colr_ref, cold_ref, rowg_ref):
    r = pl.program_id(0)
    row = mr_ref[0, 0]
    col = mr_ref[0, 1]
    self_m = row == col
    colr_ref[0] = jnp.where(self_m, TRASH, col)
    cold_ref[0] = jnp.where(self_m, NP + col, col)
    rowg_ref[0] = row + r * NP


def _prep(mr):
    out = jax.ShapeDtypeStruct((R, 6250, 128), jnp.int32)
    return pl.pallas_call(
        _prep_body,
        grid=(R,),
        in_specs=[pl.BlockSpec((1, 2, 6250, 128), lambda r: (r, 0, 0, 0))],
        out_specs=[pl.BlockSpec((1, 6250, 128), lambda r: (r, 0, 0))] * 3,
        out_shape=[out, out, out],
    )(mr)


BM2 = 2 * NP // NSUB


def _deg_body(cold, zer, deg_out, dacc, ones_v, icb, dbuf, dsem):
    c = lax.axis_index("c")
    s = lax.axis_index("s")
    for q in range(K // 16):
        ones_v[pl.ds(q * 16, 16)] = jnp.full((16,), 1.0, jnp.float32)
    pltpu.sync_copy(zer, dbuf)
    pltpu.sync_copy(dbuf, dacc.at[pl.ds(s * BM2, BM2)])
    plsc.subcore_barrier()
    wrow = (c * NSUB + s) * NC2

    def group(g, carry):
        pltpu.sync_copy(cold.at[pl.ds(wrow + g * GRP, GRP)], icb)
        for k in range(GRP):
            pltpu.async_copy(ones_v, dacc.at[icb.at[k]], dsem, add=True)
        for k in range(GRP):
            pltpu.make_async_copy(ones_v, dacc.at[icb.at[k]], dsem).wait()
        return carry

    lax.fori_loop(0, NG, group, 0)
    plsc.subcore_barrier()
    pltpu.sync_copy(dacc.at[pl.ds(s * BM2, BM2)], dbuf)
    pltpu.sync_copy(dbuf, deg_out.at[pl.ds(c * 2 * NP + s * BM2, BM2)])


_deg = functools.partial(
    pl.kernel,
    out_type=jax.ShapeDtypeStruct((R * 2 * NP,), jnp.float32),
    mesh=_mesh,
    compiler_params=_sc_params,
    scratch_types=[
        pltpu.VMEM_SHARED((2 * NP,), jnp.float32),
        pltpu.VMEM((K,), jnp.float32),
        pltpu.VMEM((GRP, K), jnp.int32),
        pltpu.VMEM((BM2,), jnp.float32),
        pltpu.SemaphoreType.DMA,
    ],
)(_deg_body)


def _mm_body(feat, w0, bb0, w1, bb1, deg, selfc,
             phi0_o, phi1_o, yq0_o, yq1_o, yq2_o, yq3_o, dis_o, diag_o):
    f = feat[...]
    p0 = jnp.dot(f, w0[0], preferred_element_type=jnp.float32) + bb0[0]
    p1 = jnp.dot(f, w1[0], preferred_element_type=jnp.float32) + bb1[0]
    loop = jnp.where(selfc[0] > 0.0, 1.0, 0.0)
    dg = deg[0] + loop
    pos = dg > 0.0
    safe = jnp.maximum(dg, 1.0)
    dis = jnp.where(pos, lax.rsqrt(safe), 0.0)
    diag = jnp.where(pos, loop / safe, 0.0)
    phi0_o[0] = p0
    phi1_o[0] = p1
    y0 = p0 * dis
    for q, ref in enumerate((yq0_o, yq1_o, yq2_o, yq3_o)):
        ref[0] = y0[:, q * 16:(q + 1) * 16]
    dis_o[0] = dis
    diag_o[0] = diag


def _mm(featp, W0, b0, W1, b1, deg, selfc):
    f64 = jax.ShapeDtypeStruct((R, NP, HID), jnp.float32)
    f16o = jax.ShapeDtypeStruct((R, NP, 16), jnp.float32)
    f1 = jax.ShapeDtypeStruct((R, NP, 1), jnp.float32)
    bs16 = pl.BlockSpec((1, BM, 16), lambda r, i: (r, i, 0))
    return pl.pallas_call(
        _mm_body,
        grid=(R, NSUB),
        in_specs=[
            pl.BlockSpec((BM, 128), lambda r, i: (i, 0)),
            pl.BlockSpec((1, 128, HID), lambda r, i: (r, 0, 0)),
            pl.BlockSpec((1, 1, HID), lambda r, i: (r, 0, 0)),
            pl.BlockSpec((1, 128, HID), lambda r, i: (r, 0, 0)),
            pl.BlockSpec((1, 1, HID), lambda r, i: (r, 0, 0)),
            pl.BlockSpec((1, BM, 1), lambda r, i: (r, i, 0)),
            pl.BlockSpec((1, BM, 1), lambda r, i: (r, i, 0)),
        ],
        out_specs=[
            pl.BlockSpec((1, BM, HID), lambda r, i: (r, i, 0)),
            pl.BlockSpec((1, BM, HID), lambda r, i: (r, i, 0)),
            bs16, bs16, bs16, bs16,
            pl.BlockSpec((1, BM, 1), lambda r, i: (r, i, 0)),
            pl.BlockSpec((1, BM, 1), lambda r, i: (r, i, 0)),
        ],
        out_shape=[f64, f64, f16o, f16o, f16o, f16o, f1, f1],
    )(featp, W0, b0, W1, b1, deg, selfc)


def _run_phases(c, s, yqs, rowg, colr, zrows, z_out, acc, igb, iscb,
                rbufs, zbuf, gsems, ssems):
    for r in range(R):
        for ph in range(2):
            pltpu.sync_copy(zrows, zbuf)
            pltpu.sync_copy(zbuf, acc.at[pl.ds(s * BM, 1568)])
            pltpu.sync_copy(zbuf.at[pl.ds(0, 1560)],
                            acc.at[pl.ds(s * BM + 1568, 1560)])
            plsc.subcore_barrier()
            wrow = (r * NSUB + s) * NC2
            ya = (yqs[0], yqs[2])[ph]
            yb = (yqs[1], yqs[3])[ph]

            def group(g, carry):
                pltpu.sync_copy(rowg.at[pl.ds(wrow + g * GRP, GRP)], igb)
                pltpu.sync_copy(colr.at[pl.ds(wrow + g * GRP, GRP)], iscb)

                def gath(k):
                    b = k % NBUF

                    @pl.when(c == 0)
                    def _():
                        pltpu.async_copy(ya.at[igb.at[k]], rbufs[b], gsems[b])

                    @pl.when(c == 1)
                    def _():
                        pltpu.async_copy(yb.at[igb.at[k]], rbufs[b], gsems[b])

                def wait_scat(k):
                    b = k % NBUF
                    pltpu.make_async_copy(
                        rbufs[b], acc.at[iscb.at[k]], ssems[b]).wait()

                for k in range(LEAD):
                    gath(k)
                for k in range(GRP):
                    b = k % NBUF
                    if k + LEAD < GRP:
                        if k >= NBUF - LEAD:
                            wait_scat(k - (NBUF - LEAD))
                        gath(k + LEAD)
                    pltpu.make_async_copy(
                        ya.at[igb.at[k]], rbufs[b], gsems[b]).wait()
                    pltpu.async_copy(rbufs[b], acc.at[iscb.at[k]],
                                     ssems[b], add=True)
                for k in range(GRP - NBUF, GRP):
                    wait_scat(k)
                return carry

            lax.fori_loop(0, NG, group, 0)
            plsc.subcore_barrier()
            qq = 2 * ph + c
            zbase = (r * 4 + qq) * NP + s * BM
            pltpu.sync_copy(acc.at[pl.ds(s * BM, 1568)], zbuf)
            pltpu.sync_copy(zbuf, z_out.at[pl.ds(zbase, 1568)])
            pltpu.sync_copy(acc.at[pl.ds(s * BM + 1568, 1560)],
                            zbuf.at[pl.ds(0, 1560)])
            pltpu.sync_copy(zbuf.at[pl.ds(0, 1560)],
                            z_out.at[pl.ds(zbase + 1568, 1560)])
            plsc.subcore_barrier()


_PROP_SCRATCH = [
    pltpu.VMEM_SHARED((NP, 16), jnp.float32),
    pltpu.VMEM((GRP, K), jnp.int32),
    pltpu.VMEM((GRP, K), jnp.int32),
    pltpu.VMEM((K, 16), jnp.float32),
    pltpu.VMEM((K, 16), jnp.float32),
    pltpu.VMEM((K, 16), jnp.float32),
    pltpu.VMEM((K, 16), jnp.float32),
    pltpu.VMEM((1568, 16), jnp.float32),
    pltpu.SemaphoreType.DMA,
    pltpu.SemaphoreType.DMA,
    pltpu.SemaphoreType.DMA,
    pltpu.SemaphoreType.DMA,
    pltpu.SemaphoreType.DMA,
    pltpu.SemaphoreType.DMA,
    pltpu.SemaphoreType.DMA,
    pltpu.SemaphoreType.DMA,
]


def _prop_body(yq0, yq1, yq2, yq3, rowg, colr, zrows, z_out,
               acc, igb, iscb, rb0, rb1, rb2, rb3, zbuf,
               gs0, gs1, gs2, gs3, ss0, ss1, ss2, ss3):
    c = lax.axis_index("c")
    s = lax.axis_index("s")
    _run_phases(c, s, (yq0, yq1, yq2, yq3), rowg, colr, zrows, z_out,
                acc, igb, iscb, (rb0, rb1, rb2, rb3), zbuf,
                (gs0, gs1, gs2, gs3), (ss0, ss1, ss2, ss3))


_prop = functools.partial(
    pl.kernel,
    out_type=jax.ShapeDtypeStruct((R * 4 * NP, 16), jnp.float32),
    mesh=_mesh,
    compiler_params=_sc_params,
    scratch_types=list(_PROP_SCRATCH),
)(_prop_body)


def _prop2f_body(yq0, yq1, yq2, yq3, rowg, colr, zrows,
                 uq0, uq1, uq2, uq3, disf, bn, z_out, og,
                 acc, igb, iscb, rb0, rb1, rb2, rb3, zbuf,
                 bnb, xib, zib0, zib1, zg0, zg1, ug0, ug1, ag, ot0, ot1,
                 gs0, gs1, gs2, gs3, ss0, ss1, ss2, ss3):
    c = lax.axis_index("c")
    s = lax.axis_index("s")
    _run_phases(c, s, (yq0, yq1, yq2, yq3), rowg, colr, zrows, z_out,
                acc, igb, iscb, (rb0, rb1, rb2, rb3), zbuf,
                (gs0, gs1, gs2, gs3), (ss0, ss1, ss2, ss3))

    def tail(uA, uB, qA, qB):
        for r in range(R):
            for cb in range(2):
                base = (s * 2 + cb) * 128
                pltpu.sync_copy(bn.at[pl.ds(base, 128)], bnb)
                for t in range(8):
                    sl = pl.ds(t * 16, 16)
                    bv = bnb[sl]
                    xib[sl] = bv + r * NP
                    zib0[sl] = bv + (r * 4 + qA) * NP
                    zib1[sl] = bv + (r * 4 + qB) * NP
                d0 = pltpu.async_copy(z_out.at[zib0], zg0, gs0)
                d1 = pltpu.async_copy(z_out.at[zib1], zg1, gs1)
                d2 = pltpu.async_copy(uA.at[xib], ug0, gs2)
                d3 = pltpu.async_copy(uB.at[xib], ug1, gs3)
                d4 = pltpu.async_copy(disf.at[xib], ag, ss0)
                d0.wait()
                d1.wait()
                d2.wait()
                d3.wait()
                d4.wait()

                def rowblk(t, carry):
                    av = C2 * ag[pl.ds(t * 16, 16)]
                    for j in range(16):
                        i = t * 16 + j
                        aj = jnp.broadcast_to(av[j], (16,))
                        ot0[i, :] = aj * zg0[i, :] + ug0[i, :]
                        ot1[i, :] = aj * zg1[i, :] + ug1[i, :]
                    return carry

                lax.fori_loop(0, 8, rowblk, 0)
                pltpu.sync_copy(ot0, og.at[pl.ds((r * 4 + qA) * B + base, 128)])
                pltpu.sync_copy(ot1, og.at[pl.ds((r * 4 + qB) * B + base, 128)])

    @pl.when(c == 0)
    def _():
        tail(uq0, uq2, 0, 2)

    @pl.when(c == 1)
    def _():
        tail(uq1, uq3, 1, 3)


_prop2f = functools.partial(
    pl.kernel,
    out_type=[jax.ShapeDtypeStruct((R * 4 * NP, 16), jnp.float32),
              jax.ShapeDtypeStruct((R * 4 * B, 16), jnp.float32)],
    mesh=_mesh,
    compiler_params=_sc_params,
    scratch_types=list(_PROP_SCRATCH[:8]) + [
        pltpu.VMEM((128,), jnp.int32),
        pltpu.VMEM((128,), jnp.int32),
        pltpu.VMEM((128,), jnp.int32),
        pltpu.VMEM((128,), jnp.int32),
        pltpu.VMEM((128, 16), jnp.float32),
        pltpu.VMEM((128, 16), jnp.float32),
        pltpu.VMEM((128, 16), jnp.float32),
        pltpu.VMEM((128, 16), jnp.float32),
        pltpu.VMEM((128,), jnp.float32),
        pltpu.VMEM((128, 16), jnp.float32),
        pltpu.VMEM((128, 16), jnp.float32),
    ] + list(_PROP_SCRATCH[8:]),
)(_prop2f_body)


def _dense1_body(phi0, phi1, dis, diag, zq0, zq1, zq2, zq3,
                 y1q0_o, y1q1_o, y1q2_o, y1q3_o,
                 uq0_o, uq1_o, uq2_o, uq3_o):
    z0 = jnp.concatenate([zq0[0], zq1[0], zq2[0], zq3[0]], axis=1)
    x = DT * phi1[0] + phi0[0] + C1 * (dis[0] * z0 + diag[0] * phi0[0])
    y1 = dis[0] * x
    u = (2.0 + C2 * diag[0]) * x - phi0[0]
    for q, ref in enumerate((y1q0_o, y1q1_o, y1q2_o, y1q3_o)):
        ref[0] = y1[:, q * 16:(q + 1) * 16]
    for q, ref in enumerate((uq0_o, uq1_o, uq2_o, uq3_o)):
        ref[0] = u[:, q * 16:(q + 1) * 16]


def _dense1(phi0, phi1, dis, diag, zq):
    f16o = jax.ShapeDtypeStruct((R, NP, 16), jnp.float32)
    bs64 = pl.BlockSpec((1, BM, HID), lambda r, i: (r, i, 0))
    bs16 = pl.BlockSpec((1, BM, 16), lambda r, i: (r, i, 0))
    bs1 = pl.BlockSpec((1, BM, 1), lambda r, i: (r, i, 0))
    return pl.pallas_call(
        _dense1_body,
        grid=(R, NSUB),
        in_specs=[bs64, bs64, bs1, bs1, bs16, bs16, bs16, bs16],
        out_specs=[bs16] * 8,
        out_shape=[f16o] * 8,
    )(phi0, phi1, dis, diag, zq[:, 0], zq[:, 1], zq[:, 2], zq[:, 3])


def _chunkify(a, padval):
    a = a.reshape(R, NSUB, 50000)
    a = jnp.pad(a, ((0, 0), (0, 0), (0, ESUB - 50000)), constant_values=padval)
    return a.reshape(R * NSUB * NC2, K)


def kernel(features, multi_r_data, batch_nodes, W0, b0, W1, b1):
    featp = jnp.pad(features, ((0, NP - N), (0, 0)))
    mr = multi_r_data.reshape(R, 2, 6250, 128)
    colr, cold, rowg = _prep(mr)
    colr_c = _chunkify(colr, TRASH)
    cold_c = _chunkify(cold, TRASH)
    rowg_c = _chunkify(rowg, 0)

    zeros2 = jnp.zeros((BM2,), jnp.float32)
    zrows = jnp.zeros((1568, 16), jnp.float32)

    dv = _deg(cold_c, zeros2).reshape(R, 2, NP)
    phi0, phi1, y0q0, y0q1, y0q2, y0q3, dis, diag = _mm(
        featp, W0, b0.reshape(R, 1, HID), W1, b1.reshape(R, 1, HID),
        dv[:, 0].reshape(R, NP, 1), dv[:, 1].reshape(R, NP, 1))

    z0 = _prop(y0q0.reshape(R * NP, 16), y0q1.reshape(R * NP, 16),
               y0q2.reshape(R * NP, 16), y0q3.reshape(R * NP, 16),
               rowg_c, colr_c, zrows).reshape(R, 4, NP, 16)
    y1q0, y1q1, y1q2, y1q3, uq0, uq1, uq2, uq3 = _dense1(
        phi0, phi1, dis, diag, z0)
    _, og = _prop2f(y1q0.reshape(R * NP, 16), y1q1.reshape(R * NP, 16),
                    y1q2.reshape(R * NP, 16), y1q3.reshape(R * NP, 16),
                    rowg_c, colr_c, zrows,
                    uq0.reshape(R * NP, 16), uq1.reshape(R * NP, 16),
                    uq2.reshape(R * NP, 16), uq3.reshape(R * NP, 16),
                    dis.reshape(R * NP), batch_nodes)
    og = og.reshape(R, 4, B, 16)
    return jnp.transpose(og, (2, 0, 1, 3)).reshape(B, R * HID)
